# Initial kernel scaffold; baseline (speedup 1.0000x reference)
import jax
import jax.numpy as jnp
from jax.experimental import pallas as pl


def kernel(token_ids, weight):
    raise NotImplementedError("write your pallas kernel here")


# speedup vs baseline: 1.6854x; 1.6854x over previous
import functools

import jax
import jax.numpy as jnp
from jax import lax
from jax.experimental import pallas as pl
from jax.experimental.pallas import tpu as pltpu
from jax.experimental.pallas import tpu_sc as plsc

_CHUNK = 128


@functools.cache
def _make_gather(num_chunks_total: int, dim: int):
    info = plsc.get_sparse_core_info()
    ncores, nsub = info.num_cores, info.num_subcores
    nw = ncores * nsub
    chunks_per_w = num_chunks_total // nw

    mesh = plsc.VectorSubcoreMesh(core_axis_name="c", subcore_axis_name="s")

    @functools.partial(
        pl.kernel,
        mesh=mesh,
        compiler_params=pltpu.CompilerParams(use_tc_tiling_on_sc=False),
        out_type=jax.ShapeDtypeStruct((num_chunks_total * _CHUNK, dim), jnp.float32),
        scratch_types=[
            pltpu.VMEM((chunks_per_w, _CHUNK), jnp.int32),
            pltpu.VMEM((2, _CHUNK, dim), jnp.float32),
            pltpu.SemaphoreType.DMA,
            pltpu.SemaphoreType.DMA,
        ],
    )
    def emb(idx_hbm, table_hbm, out_hbm, idx_v, rows_v, gsem, osem):
        wid = lax.axis_index("s") * ncores + lax.axis_index("c")
        chunk0 = wid * chunks_per_w
        pltpu.sync_copy(idx_hbm.at[pl.ds(chunk0, chunks_per_w)], idx_v)

        def step(j, _):
            pltpu.async_copy(table_hbm.at[idx_v.at[j]], rows_v.at[0], gsem).wait()
            pltpu.async_copy(
                rows_v.at[0],
                out_hbm.at[pl.ds((chunk0 + j) * _CHUNK, _CHUNK)],
                osem,
            ).wait()
            return 0

        lax.fori_loop(0, chunks_per_w, step, 0)

    return emb


def kernel(token_ids, weight):
    shape = token_ids.shape
    dim = weight.shape[1]
    flat = token_ids.reshape(-1).astype(jnp.int32)
    n = flat.shape[0]
    block = _CHUNK * 32
    pad = (-n) % block
    if pad:
        flat = jnp.concatenate([flat, jnp.zeros((pad,), jnp.int32)])
    num_chunks = (n + pad) // _CHUNK
    idx2d = flat.reshape(num_chunks, _CHUNK)
    out = _make_gather(num_chunks, dim)(idx2d, weight)
    if pad:
        out = out[:n]
    return out.reshape(*shape, dim)


# speedup vs baseline: 1.8764x; 1.1133x over previous
import functools

import jax
import jax.numpy as jnp
from jax import lax
from jax.experimental import pallas as pl
from jax.experimental.pallas import tpu as pltpu
from jax.experimental.pallas import tpu_sc as plsc

_CHUNK = 128
_NBUF = 8


@functools.cache
def _make_gather(num_chunks_total: int, dim: int):
    info = plsc.get_sparse_core_info()
    ncores, nsub = info.num_cores, info.num_subcores
    nw = ncores * nsub
    chunks_per_w = num_chunks_total // nw

    mesh = plsc.VectorSubcoreMesh(core_axis_name="c", subcore_axis_name="s")

    @functools.partial(
        pl.kernel,
        mesh=mesh,
        compiler_params=pltpu.CompilerParams(use_tc_tiling_on_sc=False),
        out_type=jax.ShapeDtypeStruct((num_chunks_total * _CHUNK, dim), jnp.float32),
        scratch_types=[
            pltpu.VMEM((chunks_per_w, _CHUNK), jnp.int32),
            pltpu.VMEM((_NBUF, _CHUNK, dim), jnp.float32),
            pltpu.SemaphoreType.DMA,
        ]
        + [pltpu.SemaphoreType.DMA] * _NBUF,
    )
    def emb(idx_hbm, table_hbm, out_hbm, idx_v, rows_v, osem, *gsems):
        wid = lax.axis_index("s") * ncores + lax.axis_index("c")
        chunk0 = wid * chunks_per_w
        pltpu.sync_copy(idx_hbm.at[pl.ds(chunk0, chunks_per_w)], idx_v)

        def gather(j, b):
            return pltpu.async_copy(
                table_hbm.at[idx_v.at[j]], rows_v.at[b], gsems[b]
            )

        def gather_wait(b):
            pltpu.make_async_copy(
                table_hbm.at[pl.ds(0, _CHUNK)], rows_v.at[b], gsems[b]
            ).wait()

        def store(j, b):
            return pltpu.async_copy(
                rows_v.at[b],
                out_hbm.at[pl.ds((chunk0 + j) * _CHUNK, _CHUNK)],
                osem,
            )

        for b in range(_NBUF):
            gather(b, b)

        @pl.loop(0, chunks_per_w - _NBUF, step=_NBUF)
        def _(g):
            for b in range(_NBUF):
                j = g + b
                gather_wait(b)
                store(j, b).wait()
                gather(j + _NBUF, b)

        for b in range(_NBUF):
            j = chunks_per_w - _NBUF + b
            gather_wait(b)
            store(j, b).wait()

    return emb


def kernel(token_ids, weight):
    shape = token_ids.shape
    dim = weight.shape[1]
    flat = token_ids.reshape(-1).astype(jnp.int32)
    n = flat.shape[0]
    block = _CHUNK * 32
    pad = (-n) % block
    if pad:
        flat = jnp.concatenate([flat, jnp.zeros((pad,), jnp.int32)])
    num_chunks = (n + pad) // _CHUNK
    idx2d = flat.reshape(num_chunks, _CHUNK)
    out = _make_gather(num_chunks, dim)(idx2d, weight)
    if pad:
        out = out[:n]
    return out.reshape(*shape, dim)
